# baseline (speedup 1.0000x reference)
import functools

import jax
import jax.numpy as jnp
import numpy as np
from jax import lax
from jax.experimental import pallas as pl
from jax.experimental.pallas import tpu as pltpu
from jax.experimental.pallas import tpu_sc as plsc

TOKENS = 8192
HIDDEN = 2048
NUM_EXPERTS = 64
TOP_K = 8
AUX_COEF = 0.01
Z_COEF = 0.001

BLOCK_T = 1024
NEG = -3.0e38


@functools.lru_cache(maxsize=1)
def _noise_np():
    with jax.ensure_compile_time_eval():
        return np.asarray(
            jax.random.normal(jax.random.key(42), (TOKENS, NUM_EXPERTS), dtype=jnp.float32)
        )


def _noise_const():
    try:
        return _noise_np()
    except Exception:
        return jax.random.normal(jax.random.key(42), (TOKENS, NUM_EXPERTS), dtype=jnp.float32)


def _logits_kernel(x_ref, w_ref, noise_ref, logits_ref, zsum_ref, acc_ref, nblocks):
    i = pl.program_id(0)
    logits_all = jnp.dot(x_ref[...], w_ref[...], preferred_element_type=jnp.float32)
    clean = logits_all[:, :NUM_EXPERTS]
    raw_noise = logits_all[:, NUM_EXPERTS:]
    stddev = jax.nn.softplus(raw_noise) + 1e-10
    logits_ref[...] = clean + noise_ref[...] * stddev

    cmax = jnp.max(clean, axis=-1, keepdims=True)
    lse = jnp.log(jnp.sum(jnp.exp(clean - cmax), axis=-1, keepdims=True)) + cmax
    z_partial = jnp.reshape(jnp.sum(lse * lse), (1, 1))

    @pl.when(i == 0)
    def _():
        acc_ref[...] = jnp.zeros_like(acc_ref)

    acc_ref[...] += z_partial

    @pl.when(i == nblocks - 1)
    def _():
        zsum_ref[...] = acc_ref[...]


def _tc_logits(x, w_cat, noise):
    nblocks = TOKENS // BLOCK_T
    return pl.pallas_call(
        functools.partial(_logits_kernel, nblocks=nblocks),
        grid=(nblocks,),
        in_specs=[
            pl.BlockSpec((BLOCK_T, HIDDEN), lambda i: (i, 0)),
            pl.BlockSpec((HIDDEN, 2 * NUM_EXPERTS), lambda i: (0, 0)),
            pl.BlockSpec((BLOCK_T, NUM_EXPERTS), lambda i: (i, 0)),
        ],
        out_specs=[
            pl.BlockSpec((BLOCK_T, NUM_EXPERTS), lambda i: (i, 0)),
            pl.BlockSpec((1, 1), lambda i: (0, 0)),
        ],
        out_shape=[
            jax.ShapeDtypeStruct((TOKENS, NUM_EXPERTS), jnp.float32),
            jax.ShapeDtypeStruct((1, 1), jnp.float32),
        ],
        scratch_shapes=[pltpu.VMEM((1, 1), jnp.float32)],
        compiler_params=pltpu.CompilerParams(dimension_semantics=("arbitrary",)),
    )(x, w_cat, noise)


NW = 32
ROWS_W = TOKENS // NW
VPR = NUM_EXPERTS // 16


def _bfly(x, op):
    lanes = lax.iota(jnp.int32, 16)
    for k in (1, 2, 4, 8):
        x = op(x, x.at[lanes ^ k].get(mode="promise_in_bounds"))
    return x


def _sc_route(logits):
    logits3 = logits.reshape(NW, ROWS_W * VPR, 16)
    mesh = plsc.VectorSubcoreMesh(core_axis_name="c", subcore_axis_name="s")

    @functools.partial(
        pl.kernel,
        mesh=mesh,
        out_type=[
            jax.ShapeDtypeStruct((NW, ROWS_W * VPR, 16), jnp.float32),
            jax.ShapeDtypeStruct((NW, 2 * VPR, 16), jnp.float32),
        ],
        scratch_types=[
            pltpu.VMEM((ROWS_W * VPR, 16), jnp.float32),
            pltpu.VMEM((ROWS_W * VPR, 16), jnp.float32),
            pltpu.VMEM((2 * VPR, 16), jnp.float32),
        ],
        compiler_params=pltpu.CompilerParams(use_tc_tiling_on_sc=False),
    )
    def k(logits_hbm, gates_hbm, stats_hbm, logits_v, gates_v, stats_v):
        wid = lax.axis_index("s") * 2 + lax.axis_index("c")
        pltpu.sync_copy(logits_hbm.at[wid], logits_v)

        def row_body(r, carry):
            accs = list(carry)
            v = [logits_v[r * VPR + j, :] for j in range(VPR)]
            running = list(v)
            rowmax = None
            thresh = None
            for it in range(TOP_K):
                m01 = jnp.maximum(running[0], running[1])
                m23 = jnp.maximum(running[2], running[3])
                mall = _bfly(jnp.maximum(m01, m23), jnp.maximum)
                if it == 0:
                    rowmax = mall
                running = [jnp.where(rj == mall, NEG, rj) for rj in running]
                thresh = mall
            e = [jnp.exp(vj - rowmax) for vj in v]
            sel_e = [jnp.where(vj >= thresh, ej, 0.0) for vj, ej in zip(v, e)]
            denom = _bfly(sel_e[0] + sel_e[1] + sel_e[2] + sel_e[3],
                          lambda a, b: a + b)
            for j in range(VPR):
                g = sel_e[j] / denom
                gates_v[r * VPR + j, :] = g
                accs[j] = accs[j] + g
                accs[VPR + j] = accs[VPR + j] + jnp.where(g > 0.0, 1.0, 0.0)
            return tuple(accs)

        zero = jnp.zeros((16,), jnp.float32)
        accs = lax.fori_loop(0, ROWS_W, row_body, (zero,) * (2 * VPR))
        for j in range(2 * VPR):
            stats_v[j, :] = accs[j]
        pltpu.sync_copy(gates_v, gates_hbm.at[wid])
        pltpu.sync_copy(stats_v, stats_hbm.at[wid])

    gates3, stats = k(logits3)
    return gates3.reshape(TOKENS, NUM_EXPERTS), stats


def kernel(x, w_gate, w_noise):
    w_cat = jnp.concatenate([w_gate, w_noise], axis=0).T
    noise = jnp.asarray(_noise_const())
    logits, zsum = _tc_logits(x, w_cat, noise)
    gates, stats = _sc_route(logits)
    s = jnp.sum(stats, axis=0)
    imp = s[:VPR].reshape(-1)
    loads = s[VPR:].reshape(-1)
    lb = AUX_COEF * (NUM_EXPERTS * jnp.sum(imp * loads) / float(TOKENS * TOKENS))
    zl = Z_COEF * zsum[0, 0] / float(TOKENS)
    return gates, lb + zl


# speedup vs baseline: 2.0715x; 2.0715x over previous
import functools

import jax
import jax.numpy as jnp
import numpy as np
from jax.experimental import pallas as pl
from jax.experimental.pallas import tpu as pltpu

TOKENS = 8192
HIDDEN = 2048
NUM_EXPERTS = 64
TOP_K = 8
AUX_COEF = 0.01
Z_COEF = 0.001

BLOCK_T = 1024


@functools.lru_cache(maxsize=1)
def _noise_np():
    with jax.ensure_compile_time_eval():
        return np.asarray(
            jax.random.normal(jax.random.key(42), (TOKENS, NUM_EXPERTS), dtype=jnp.float32)
        )


def _noise_const():
    try:
        return _noise_np()
    except Exception:
        return jax.random.normal(jax.random.key(42), (TOKENS, NUM_EXPERTS), dtype=jnp.float32)


NCHUNKS = 4


def _route_chunk(logits, clean, gates_ref, row0, bt):
    running = logits
    for j in range(TOP_K):
        m = jnp.max(running, axis=-1, keepdims=True)
        if j == 0:
            rowmax = m
        running = jnp.where(running == m, -jnp.inf, running)
        thresh = m

    sel = logits >= thresh
    e = jnp.exp(logits - rowmax)
    sel_e = jnp.where(sel, e, 0.0)
    denom = jnp.sum(sel_e, axis=-1, keepdims=True)
    gates = sel_e / denom
    gates_ref[pl.ds(row0, bt), :] = gates

    imp_partial = jnp.sum(gates, axis=0, keepdims=True)
    loads_partial = jnp.sum((gates > 0.0).astype(jnp.float32), axis=0, keepdims=True)
    stats = jnp.concatenate([imp_partial, loads_partial], axis=1)

    cmax = jnp.max(clean, axis=-1, keepdims=True)
    lse = jnp.log(jnp.sum(jnp.exp(clean - cmax), axis=-1, keepdims=True)) + cmax
    z_partial = jnp.reshape(jnp.sum(lse * lse), (1, 1))
    return stats, z_partial


def _router_kernel(x_ref, wg_ref, wn_ref, noise_ref, gates_ref, loss_ref,
                   acc_ref, wt_ref, nblocks):
    i = pl.program_id(0)

    @pl.when(i == 0)
    def _():
        wt_ref[:, :NUM_EXPERTS] = wg_ref[...].T
        wt_ref[:, NUM_EXPERTS:] = wn_ref[...].T

    w = wt_ref[...]
    ct = BLOCK_T // NCHUNKS

    chunks = []
    for c in range(NCHUNKS):
        r0 = c * ct
        logits_all = jnp.dot(x_ref[pl.ds(r0, ct), :], w,
                             preferred_element_type=jnp.float32)
        clean = logits_all[:, :NUM_EXPERTS]
        raw_noise = logits_all[:, NUM_EXPERTS:]
        stddev = jax.nn.softplus(raw_noise) + 1e-10
        logits = clean + noise_ref[pl.ds(r0, ct), :] * stddev
        chunks.append((logits, clean, r0))

    stats = jnp.zeros((1, 2 * NUM_EXPERTS), jnp.float32)
    z_partial = jnp.zeros((1, 1), jnp.float32)
    for logits, clean, r0 in chunks:
        s, z = _route_chunk(logits, clean, gates_ref, r0, ct)
        stats = stats + s
        z_partial = z_partial + z

    @pl.when(i == 0)
    def _():
        acc_ref[...] = jnp.zeros_like(acc_ref)

    acc_ref[0:1, :] += stats
    acc_ref[1:2, 0:1] += z_partial

    @pl.when(i == nblocks - 1)
    def _():
        imp = acc_ref[0:1, :NUM_EXPERTS]
        loads = acc_ref[0:1, NUM_EXPERTS:]
        zsum = acc_ref[1:2, 0:1]
        lb = AUX_COEF * (NUM_EXPERTS * jnp.sum(imp * loads) / float(TOKENS * TOKENS))
        zl = Z_COEF * zsum[0, 0] / float(TOKENS)
        loss_ref[...] = jnp.reshape(lb + zl, (1, 1))


def kernel(x, w_gate, w_noise):
    noise = jnp.asarray(_noise_const())
    nblocks = TOKENS // BLOCK_T

    gates, loss = pl.pallas_call(
        functools.partial(_router_kernel, nblocks=nblocks),
        grid=(nblocks,),
        in_specs=[
            pl.BlockSpec((BLOCK_T, HIDDEN), lambda i: (i, 0)),
            pl.BlockSpec((NUM_EXPERTS, HIDDEN), lambda i: (0, 0)),
            pl.BlockSpec((NUM_EXPERTS, HIDDEN), lambda i: (0, 0)),
            pl.BlockSpec((BLOCK_T, NUM_EXPERTS), lambda i: (i, 0)),
        ],
        out_specs=[
            pl.BlockSpec((BLOCK_T, NUM_EXPERTS), lambda i: (i, 0)),
            pl.BlockSpec((1, 1), lambda i: (0, 0)),
        ],
        out_shape=[
            jax.ShapeDtypeStruct((TOKENS, NUM_EXPERTS), jnp.float32),
            jax.ShapeDtypeStruct((1, 1), jnp.float32),
        ],
        scratch_shapes=[
            pltpu.VMEM((8, 2 * NUM_EXPERTS), jnp.float32),
            pltpu.VMEM((HIDDEN, 2 * NUM_EXPERTS), jnp.float32),
        ],
        compiler_params=pltpu.CompilerParams(
            dimension_semantics=("arbitrary",),
        ),
    )(x, w_gate, w_noise, noise)
    return gates, jnp.reshape(loss, ())
